# Initial kernel scaffold; baseline (speedup 1.0000x reference)
import jax
import jax.numpy as jnp
from jax.experimental import pallas as pl


def kernel(x, edge):
    raise NotImplementedError("write your pallas kernel here")


# speedup vs baseline: 9.0336x; 9.0336x over previous
import functools

import jax
import jax.numpy as jnp
from jax import lax
from jax.experimental import pallas as pl
from jax.experimental.pallas import tpu as pltpu
from jax.experimental.pallas import tpu_sc as plsc

NSUB = 16
CHUNK = 128
PADCOL = 16


def _sc_pass(n_pad, n_cols, e_pad, dtype):
  ept = e_pad // NSUB
  nch = ept // CHUNK
  rps = n_pad // NSUB

  def body(tab_a, gidx_a, sidx_a, tab_b, gidx_b, sidx_b, zinit,
           out_a, out_b, idxg, idxs, rows, acc, semg):
    c = lax.axis_index("c")
    s = lax.axis_index("s")
    srow = pl.multiple_of(s * rps, 8)
    pltpu.sync_copy(zinit.at[pl.ds(srow, rps)],
                    acc.at[pl.ds(srow, rps)])
    plsc.subcore_barrier()

    def chunk(i, carry):
      b = s * ept + i * CHUNK

      @pl.when(c == 0)
      def _():
        pltpu.sync_copy(gidx_a.at[pl.ds(b, CHUNK)], idxg)
        pltpu.sync_copy(sidx_a.at[pl.ds(b, CHUNK)], idxs)
        pltpu.async_copy(tab_a.at[idxg], rows, semg).wait()

      @pl.when(c == 1)
      def _():
        pltpu.sync_copy(gidx_b.at[pl.ds(b, CHUNK)], idxg)
        pltpu.sync_copy(sidx_b.at[pl.ds(b, CHUNK)], idxs)
        pltpu.async_copy(tab_b.at[idxg], rows, semg).wait()

      pltpu.sync_copy(rows, acc.at[idxs], add=True)
      return carry

    lax.fori_loop(0, nch, chunk, 0)
    plsc.subcore_barrier()

    @pl.when(c == 0)
    def _():
      pltpu.sync_copy(acc.at[pl.ds(srow, rps)],
                      out_a.at[pl.ds(srow, rps)])

    @pl.when(c == 1)
    def _():
      pltpu.sync_copy(acc.at[pl.ds(srow, rps)],
                      out_b.at[pl.ds(srow, rps)])

  shape = jax.ShapeDtypeStruct((n_pad, n_cols), dtype)
  return pl.kernel(
      body,
      out_type=[shape, shape],
      mesh=plsc.VectorSubcoreMesh(core_axis_name="c", subcore_axis_name="s"),
      scratch_types=[
          pltpu.VMEM((CHUNK,), jnp.int32),
          pltpu.VMEM((CHUNK,), jnp.int32),
          pltpu.VMEM((CHUNK, n_cols), dtype),
          pltpu.VMEM_SHARED((n_pad, n_cols), dtype),
          pltpu.SemaphoreType.DMA,
      ],
      compiler_params=pltpu.CompilerParams(use_tc_tiling_on_sc=False),
  )


def _scaled(acc, d):
  deg = acc[:, d:d + 1]
  return jnp.maximum(acc[:, :d] / jnp.maximum(deg, 1.0), 0.0)


def _l2n(v):
  n = jnp.sqrt(jnp.sum(v * v, axis=1, keepdims=True))
  return v / jnp.maximum(n, 1e-12)


def _marker_cols(rows, dtype):
  col = lax.broadcasted_iota(jnp.int32, (rows, PADCOL), 1)
  return jnp.where(col == 0, jnp.array(1.0, dtype), jnp.array(0.0, dtype))


def _tc1_body(d, acc_e_ref, acc_h_ref, xe_ref, g1_ref):
  rows = xe_ref.shape[0]
  dtype = xe_ref.dtype
  mark = _marker_cols(rows, dtype)
  xe_ref[:, :d] = _scaled(acc_e_ref[...], d)
  xe_ref[:, d:] = mark
  g1_ref[:, :d] = _l2n(_scaled(acc_h_ref[...], d))
  g1_ref[:, d:] = mark


def _tc2_body(d, acc_v_ref, acc_h2_ref, x_ref, g1_ref, out_ref):
  x_v = _l2n(_scaled(acc_v_ref[...], d))
  h2 = _l2n(_scaled(acc_h2_ref[...], d))
  cat = jnp.concatenate([x_ref[...], x_v, g1_ref[:, :d], h2], axis=1)
  out_ref[...] = _l2n(cat)


def kernel(x, edge):
  n, d = x.shape
  e = edge.shape[1]
  dtype = x.dtype
  n_cols = d + PADCOL
  n_pad = ((n + 1 + NSUB * 8 - 1) // (NSUB * 8)) * (NSUB * 8)
  e_pad = ((e + NSUB * CHUNK - 1) // (NSUB * CHUNK)) * (NSUB * CHUNK)

  xp = jnp.zeros((n_pad, n_cols), dtype)
  xp = xp.at[:n, :d].set(x)
  xp = xp.at[:n, d].set(1.0)

  pad = jnp.full((e_pad - e,), n, jnp.int32)
  src = jnp.concatenate([edge[0], pad])
  eid = jnp.concatenate([edge[1], pad])
  dst = jnp.concatenate([edge[2], pad])
  zinit = jnp.zeros((n_pad, n_cols), dtype)

  sc = _sc_pass(n_pad, n_cols, e_pad, dtype)

  acc_e, acc_h1 = sc(xp, src, eid, xp, src, dst, zinit)

  br = max(b for b in range(8, n_pad + 1, 8)
           if n_pad % b == 0 and b * n_cols * 4 <= 1536 * 1024)
  grid = (n_pad // br,)
  spec = pl.BlockSpec((br, n_cols), lambda i: (i, 0))
  xe, g1 = pl.pallas_call(
      functools.partial(_tc1_body, d),
      grid=grid,
      in_specs=[spec, spec],
      out_specs=[spec, spec],
      out_shape=[jax.ShapeDtypeStruct((n_pad, n_cols), dtype)] * 2,
  )(acc_e, acc_h1)

  acc_v, acc_h2 = sc(xe, eid, src, g1, src, dst, zinit)

  x2 = jnp.zeros((n_pad, d), dtype).at[:n].set(x)
  xspec = pl.BlockSpec((br, d), lambda i: (i, 0))
  ospec = pl.BlockSpec((br, 4 * d), lambda i: (i, 0))
  out = pl.pallas_call(
      functools.partial(_tc2_body, d),
      grid=grid,
      in_specs=[spec, spec, xspec, spec],
      out_specs=ospec,
      out_shape=jax.ShapeDtypeStruct((n_pad, 4 * d), dtype),
  )(acc_v, acc_h2, x2, g1)
  return out[:n]


# speedup vs baseline: 11.5715x; 1.2809x over previous
import functools

import jax
import jax.numpy as jnp
from jax import lax
from jax.experimental import pallas as pl
from jax.experimental.pallas import tpu as pltpu
from jax.experimental.pallas import tpu_sc as plsc

NSUB = 16
CHUNK = 64
PADCOL = 16
NBUF = 3


def _sc_pass(n_pad, n_cols, e_pad, dtype):
  ept = e_pad // NSUB
  nch = ept // CHUNK
  rps = n_pad // NSUB

  def body(tab_a, idx_a, tab_b, idx_b, zinit,
           out_a, out_b, idx2, rows, acc, *sems):
    semi = sems[:NBUF]
    semg = sems[NBUF:]
    c = lax.axis_index("c")
    s = lax.axis_index("s")
    srow = pl.multiple_of(s * rps, 8)
    pltpu.sync_copy(zinit.at[pl.ds(srow, rps)],
                    acc.at[pl.ds(srow, rps)])

    def start_idx(i, b):
      cid = s * nch + i

      @pl.when(c == 0)
      def _():
        pltpu.async_copy(idx_a.at[cid], idx2.at[b], semi[b])

      @pl.when(c == 1)
      def _():
        pltpu.async_copy(idx_b.at[cid], idx2.at[b], semi[b])

    def wait_idx(b):
      pltpu.make_async_copy(idx_a.at[0], idx2.at[b], semi[b]).wait()

    def start_gather(b):
      @pl.when(c == 0)
      def _():
        pltpu.async_copy(tab_a.at[idx2.at[b, 0]], rows.at[b], semg[b])

      @pl.when(c == 1)
      def _():
        pltpu.async_copy(tab_b.at[idx2.at[b, 0]], rows.at[b], semg[b])

    def wait_gather(b):
      pltpu.make_async_copy(tab_a.at[idx2.at[b, 0]], rows.at[b],
                            semg[b]).wait()

    for b in range(NBUF):
      start_idx(b, b)
    wait_idx(0)
    start_gather(0)
    plsc.subcore_barrier()

    def group(g, carry):
      for b in range(NBUF):
        i = g * NBUF + b
        nb = (b + 1) % NBUF

        @pl.when(i + 1 < nch)
        def _():
          wait_idx(nb)
          start_gather(nb)

        wait_gather(b)
        pltpu.sync_copy(rows.at[b], acc.at[idx2.at[b, 1]], add=True)

        @pl.when(i + NBUF < nch)
        def _():
          start_idx(i + NBUF, b)
      return carry

    lax.fori_loop(0, nch // NBUF, group, 0)
    plsc.subcore_barrier()

    @pl.when(c == 0)
    def _():
      pltpu.sync_copy(acc.at[pl.ds(srow, rps)],
                      out_a.at[pl.ds(srow, rps)])

    @pl.when(c == 1)
    def _():
      pltpu.sync_copy(acc.at[pl.ds(srow, rps)],
                      out_b.at[pl.ds(srow, rps)])

  shape = jax.ShapeDtypeStruct((n_pad, n_cols), dtype)
  return pl.kernel(
      body,
      out_type=[shape, shape],
      mesh=plsc.VectorSubcoreMesh(core_axis_name="c", subcore_axis_name="s"),
      scratch_types=[
          pltpu.VMEM((NBUF, 2, CHUNK), jnp.int32),
          pltpu.VMEM((NBUF, CHUNK, n_cols), dtype),
          pltpu.VMEM_SHARED((n_pad, n_cols), dtype),
      ] + [pltpu.SemaphoreType.DMA] * (2 * NBUF),
      compiler_params=pltpu.CompilerParams(use_tc_tiling_on_sc=False),
  )


def _scaled(acc, d):
  deg = acc[:, d:d + 1]
  return jnp.maximum(acc[:, :d] / jnp.maximum(deg, 1.0), 0.0)


def _l2n(v):
  n = jnp.sqrt(jnp.sum(v * v, axis=1, keepdims=True))
  return v / jnp.maximum(n, 1e-12)


def _marker_cols(rows, dtype):
  col = lax.broadcasted_iota(jnp.int32, (rows, PADCOL), 1)
  return jnp.where(col == 0, jnp.array(1.0, dtype), jnp.array(0.0, dtype))


def _tc1_body(d, acc_e_ref, acc_h_ref, xe_ref, g1_ref):
  rows = xe_ref.shape[0]
  dtype = xe_ref.dtype
  mark = _marker_cols(rows, dtype)
  xe_ref[:, :d] = _scaled(acc_e_ref[...], d)
  xe_ref[:, d:] = mark
  g1_ref[:, :d] = _l2n(_scaled(acc_h_ref[...], d))
  g1_ref[:, d:] = mark


def _tc2_body(d, acc_v_ref, acc_h2_ref, x_ref, g1_ref, out_ref):
  x_v = _l2n(_scaled(acc_v_ref[...], d))
  h2 = _l2n(_scaled(acc_h2_ref[...], d))
  cat = jnp.concatenate([x_ref[...], x_v, g1_ref[:, :d], h2], axis=1)
  out_ref[...] = _l2n(cat)


def kernel(x, edge):
  n, d = x.shape
  e = edge.shape[1]
  dtype = x.dtype
  n_cols = d + PADCOL
  n_pad = ((n + 1 + NSUB * 8 - 1) // (NSUB * 8)) * (NSUB * 8)
  egrp = NSUB * CHUNK * NBUF
  e_pad = ((e + egrp - 1) // egrp) * egrp

  xp = jnp.zeros((n_pad, n_cols), dtype)
  xp = xp.at[:n, :d].set(x)
  xp = xp.at[:n, d].set(1.0)

  pad = jnp.full((e_pad - e,), n, jnp.int32)
  src = jnp.concatenate([edge[0], pad])
  eid = jnp.concatenate([edge[1], pad])
  dst = jnp.concatenate([edge[2], pad])
  zinit = jnp.zeros((n_pad, n_cols), dtype)

  def pair(g, sct):
    return jnp.stack([g.reshape(-1, CHUNK), sct.reshape(-1, CHUNK)], axis=1)

  src_eid = pair(src, eid)
  src_dst = pair(src, dst)
  eid_src = pair(eid, src)

  sc = _sc_pass(n_pad, n_cols, e_pad, dtype)

  acc_e, acc_h1 = sc(xp, src_eid, xp, src_dst, zinit)

  br = max(b for b in range(8, n_pad + 1, 8)
           if n_pad % b == 0 and b * n_cols * 4 <= 1536 * 1024)
  grid = (n_pad // br,)
  spec = pl.BlockSpec((br, n_cols), lambda i: (i, 0))
  xe, g1 = pl.pallas_call(
      functools.partial(_tc1_body, d),
      grid=grid,
      in_specs=[spec, spec],
      out_specs=[spec, spec],
      out_shape=[jax.ShapeDtypeStruct((n_pad, n_cols), dtype)] * 2,
  )(acc_e, acc_h1)

  acc_v, acc_h2 = sc(xe, eid_src, g1, src_dst, zinit)

  x2 = jnp.zeros((n_pad, d), dtype).at[:n].set(x)
  xspec = pl.BlockSpec((br, d), lambda i: (i, 0))
  ospec = pl.BlockSpec((br, 4 * d), lambda i: (i, 0))
  out = pl.pallas_call(
      functools.partial(_tc2_body, d),
      grid=grid,
      in_specs=[spec, spec, xspec, spec],
      out_specs=ospec,
      out_shape=jax.ShapeDtypeStruct((n_pad, 4 * d), dtype),
  )(acc_v, acc_h2, x2, g1)
  return out[:n]


# speedup vs baseline: 12.8666x; 1.1119x over previous
import functools

import jax
import jax.numpy as jnp
from jax import lax
from jax.experimental import pallas as pl
from jax.experimental.pallas import tpu as pltpu
from jax.experimental.pallas import tpu_sc as plsc

NSUB = 16
CHUNK = 64
PADCOL = 16
NBUF = 3


def _sc_pass(n_pad, n_cols, e_pad, dtype):
  ept = e_pad // NSUB
  nch = ept // CHUNK
  rps = n_pad // NSUB

  def body(tab_a, idx_a, tab_b, idx_b, zinit,
           out_a, out_b, idx2, rows, acc, *sems):
    semi = sems[:NBUF]
    semg = sems[NBUF:]
    c = lax.axis_index("c")
    s = lax.axis_index("s")
    srow = pl.multiple_of(s * rps, 8)
    pltpu.sync_copy(zinit.at[pl.ds(srow, rps)],
                    acc.at[pl.ds(srow, rps)])

    def start_idx(i, b):
      cid = s * nch + i

      @pl.when(c == 0)
      def _():
        pltpu.async_copy(idx_a.at[cid], idx2.at[b], semi[b])

      @pl.when(c == 1)
      def _():
        pltpu.async_copy(idx_b.at[cid], idx2.at[b], semi[b])

    def wait_idx(b):
      pltpu.make_async_copy(idx_a.at[0], idx2.at[b], semi[b]).wait()

    def start_gather(b):
      @pl.when(c == 0)
      def _():
        pltpu.async_copy(tab_a.at[idx2.at[b, 0]], rows.at[b], semg[b])

      @pl.when(c == 1)
      def _():
        pltpu.async_copy(tab_b.at[idx2.at[b, 0]], rows.at[b], semg[b])

    def wait_gather(b):
      pltpu.make_async_copy(tab_a.at[idx2.at[b, 0]], rows.at[b],
                            semg[b]).wait()

    for b in range(NBUF):
      start_idx(b, b)
    wait_idx(0)
    start_gather(0)
    plsc.subcore_barrier()

    def group(g, carry):
      for b in range(NBUF):
        i = g * NBUF + b
        nb = (b + 1) % NBUF

        @pl.when(i + 1 < nch)
        def _():
          wait_idx(nb)
          start_gather(nb)

        wait_gather(b)
        pltpu.sync_copy(rows.at[b], acc.at[idx2.at[b, 1]], add=True)

        @pl.when(i + NBUF < nch)
        def _():
          start_idx(i + NBUF, b)
      return carry

    lax.fori_loop(0, nch // NBUF, group, 0)
    plsc.subcore_barrier()

    @pl.when(c == 0)
    def _():
      pltpu.sync_copy(acc.at[pl.ds(srow, rps)],
                      out_a.at[pl.ds(srow, rps)])

    @pl.when(c == 1)
    def _():
      pltpu.sync_copy(acc.at[pl.ds(srow, rps)],
                      out_b.at[pl.ds(srow, rps)])

  shape = jax.ShapeDtypeStruct((n_pad, n_cols), dtype)
  return pl.kernel(
      body,
      out_type=[shape, shape],
      mesh=plsc.VectorSubcoreMesh(core_axis_name="c", subcore_axis_name="s"),
      scratch_types=[
          pltpu.VMEM((NBUF, 2, CHUNK), jnp.int32),
          pltpu.VMEM((NBUF, CHUNK, n_cols), dtype),
          pltpu.VMEM_SHARED((n_pad, n_cols), dtype),
      ] + [pltpu.SemaphoreType.DMA] * (2 * NBUF),
      compiler_params=pltpu.CompilerParams(use_tc_tiling_on_sc=False),
  )


def _sc_pass_shared(n_pad, n_half, e_pad, dtype):
  ept = e_pad // NSUB
  nch = ept // CHUNK
  rps = n_pad // NSUB

  def body(tab_lo, tab_hi, idx3, zin,
           o_e_lo, o_e_hi, o_h_lo, o_h_hi,
           idxr, rows, acc1, acc2, *sems):
    semi = sems[:NBUF]
    semg = sems[NBUF:]
    c = lax.axis_index("c")
    s = lax.axis_index("s")
    srow = pl.multiple_of(s * rps, 8)
    pltpu.sync_copy(zin.at[pl.ds(srow, rps)], acc1.at[pl.ds(srow, rps)])
    pltpu.sync_copy(zin.at[pl.ds(srow, rps)], acc2.at[pl.ds(srow, rps)])

    def start_idx(i, b):
      pltpu.async_copy(idx3.at[s * nch + i], idxr.at[b], semi[b])

    def wait_idx(b):
      pltpu.make_async_copy(idx3.at[0], idxr.at[b], semi[b]).wait()

    def start_gather(b):
      @pl.when(c == 0)
      def _():
        pltpu.async_copy(tab_lo.at[idxr.at[b, 0]], rows.at[b], semg[b])

      @pl.when(c == 1)
      def _():
        pltpu.async_copy(tab_hi.at[idxr.at[b, 0]], rows.at[b], semg[b])

    def wait_gather(b):
      pltpu.make_async_copy(tab_lo.at[idxr.at[b, 0]], rows.at[b],
                            semg[b]).wait()

    for b in range(NBUF):
      start_idx(b, b)
    wait_idx(0)
    start_gather(0)
    plsc.subcore_barrier()

    def group(g, carry):
      for b in range(NBUF):
        i = g * NBUF + b
        nb = (b + 1) % NBUF

        @pl.when(i + 1 < nch)
        def _():
          wait_idx(nb)
          start_gather(nb)

        wait_gather(b)
        pltpu.sync_copy(rows.at[b], acc1.at[idxr.at[b, 1]], add=True)
        pltpu.sync_copy(rows.at[b], acc2.at[idxr.at[b, 2]], add=True)

        @pl.when(i + NBUF < nch)
        def _():
          start_idx(i + NBUF, b)
      return carry

    lax.fori_loop(0, nch // NBUF, group, 0)
    plsc.subcore_barrier()

    @pl.when(c == 0)
    def _():
      pltpu.sync_copy(acc1.at[pl.ds(srow, rps)], o_e_lo.at[pl.ds(srow, rps)])
      pltpu.sync_copy(acc2.at[pl.ds(srow, rps)], o_h_lo.at[pl.ds(srow, rps)])

    @pl.when(c == 1)
    def _():
      pltpu.sync_copy(acc1.at[pl.ds(srow, rps)], o_e_hi.at[pl.ds(srow, rps)])
      pltpu.sync_copy(acc2.at[pl.ds(srow, rps)], o_h_hi.at[pl.ds(srow, rps)])

  shape = jax.ShapeDtypeStruct((n_pad, n_half), dtype)
  return pl.kernel(
      body,
      out_type=[shape] * 4,
      mesh=plsc.VectorSubcoreMesh(core_axis_name="c", subcore_axis_name="s"),
      scratch_types=[
          pltpu.VMEM((NBUF, 3, CHUNK), jnp.int32),
          pltpu.VMEM((NBUF, CHUNK, n_half), dtype),
          pltpu.VMEM_SHARED((n_pad, n_half), dtype),
          pltpu.VMEM_SHARED((n_pad, n_half), dtype),
      ] + [pltpu.SemaphoreType.DMA] * (2 * NBUF),
      compiler_params=pltpu.CompilerParams(use_tc_tiling_on_sc=False),
  )


def _scaled(acc, d):
  deg = acc[:, d:d + 1]
  return jnp.maximum(acc[:, :d] / jnp.maximum(deg, 1.0), 0.0)


def _l2n(v):
  n = jnp.sqrt(jnp.sum(v * v, axis=1, keepdims=True))
  return v / jnp.maximum(n, 1e-12)


def _marker_cols(rows, dtype):
  col = lax.broadcasted_iota(jnp.int32, (rows, PADCOL), 1)
  return jnp.where(col == 0, jnp.array(1.0, dtype), jnp.array(0.0, dtype))


def _tc1_body(d, e_lo_ref, e_hi_ref, h_lo_ref, h_hi_ref, xe_ref, g1_ref):
  rows = xe_ref.shape[0]
  dtype = xe_ref.dtype
  hd = d // 2
  mark = _marker_cols(rows, dtype)

  def join(lo, hi):
    acc = jnp.concatenate([lo[:, :hd], hi[:, :hd]], axis=1)
    deg = lo[:, hd:hd + 1]
    return jnp.maximum(acc / jnp.maximum(deg, 1.0), 0.0)

  xe_ref[:, :d] = join(e_lo_ref[...], e_hi_ref[...])
  xe_ref[:, d:] = mark
  g1_ref[:, :d] = _l2n(join(h_lo_ref[...], h_hi_ref[...]))
  g1_ref[:, d:] = mark


def _tc2_body(d, acc_v_ref, acc_h2_ref, x_ref, g1_ref, out_ref):
  x_v = _l2n(_scaled(acc_v_ref[...], d))
  h2 = _l2n(_scaled(acc_h2_ref[...], d))
  cat = jnp.concatenate([x_ref[...], x_v, g1_ref[:, :d], h2], axis=1)
  out_ref[...] = _l2n(cat)


def kernel(x, edge):
  n, d = x.shape
  e = edge.shape[1]
  dtype = x.dtype
  n_cols = d + PADCOL
  n_pad = ((n + 1 + NSUB * 8 - 1) // (NSUB * 8)) * (NSUB * 8)
  egrp = NSUB * CHUNK * NBUF
  e_pad = ((e + egrp - 1) // egrp) * egrp

  hd = d // 2
  n_half = hd + PADCOL

  def half_table(cols):
    t = jnp.zeros((n_pad, n_half), dtype)
    t = t.at[:n, :hd].set(cols)
    return t.at[:n, hd].set(1.0)

  x_lo = half_table(x[:, :hd])
  x_hi = half_table(x[:, hd:])

  pad = jnp.full((e_pad - e,), n, jnp.int32)
  src = jnp.concatenate([edge[0], pad])
  eid = jnp.concatenate([edge[1], pad])
  dst = jnp.concatenate([edge[2], pad])
  zinit = jnp.zeros((n_pad, n_cols), dtype)

  def chunked(ix):
    return [a.reshape(-1, CHUNK) for a in ix]

  src_eid_dst = jnp.stack(chunked([src, eid, dst]), axis=1)
  eid_src = jnp.stack(chunked([eid, src]), axis=1)
  src_dst = jnp.stack(chunked([src, dst]), axis=1)

  sc1 = _sc_pass_shared(n_pad, n_half, e_pad, dtype)
  e_lo, e_hi, h_lo, h_hi = sc1(x_lo, x_hi, src_eid_dst, zinit[:, :n_half])

  br = max(b for b in range(8, n_pad + 1, 8)
           if n_pad % b == 0 and b * n_cols * 4 <= 1536 * 1024)
  grid = (n_pad // br,)
  spec = pl.BlockSpec((br, n_cols), lambda i: (i, 0))
  hspec = pl.BlockSpec((br, n_half), lambda i: (i, 0))
  xe, g1 = pl.pallas_call(
      functools.partial(_tc1_body, d),
      grid=grid,
      in_specs=[hspec] * 4,
      out_specs=[spec, spec],
      out_shape=[jax.ShapeDtypeStruct((n_pad, n_cols), dtype)] * 2,
  )(e_lo, e_hi, h_lo, h_hi)

  sc = _sc_pass(n_pad, n_cols, e_pad, dtype)

  acc_v, acc_h2 = sc(xe, eid_src, g1, src_dst, zinit)

  x2 = jnp.zeros((n_pad, d), dtype).at[:n].set(x)
  xspec = pl.BlockSpec((br, d), lambda i: (i, 0))
  ospec = pl.BlockSpec((br, 4 * d), lambda i: (i, 0))
  out = pl.pallas_call(
      functools.partial(_tc2_body, d),
      grid=grid,
      in_specs=[spec, spec, xspec, spec],
      out_specs=ospec,
      out_shape=jax.ShapeDtypeStruct((n_pad, 4 * d), dtype),
  )(acc_v, acc_h2, x2, g1)
  return out[:n]
